# Initial kernel scaffold; baseline (speedup 1.0000x reference)
import jax
import jax.numpy as jnp
from jax.experimental import pallas as pl


def kernel(x, edge_index, W1, b1, W2, b2):
    raise NotImplementedError("write your pallas kernel here")


# speedup vs baseline: 8.9791x; 8.9791x over previous
import functools

import jax
import jax.numpy as jnp
from jax import lax
from jax.experimental import pallas as pl
from jax.experimental.pallas import tpu as pltpu
from jax.experimental.pallas import tpu_sc as plsc

NC = 2
NS = 16
K = 128
D = 128


def _mesh():
    return plsc.VectorSubcoreMesh(
        core_axis_name="c", subcore_axis_name="s", num_cores=NC, num_subcores=NS
    )


def _agg_kernel(acc_rows, nch_w, gather):
    stripe = acc_rows // NS
    scratch = [
        pltpu.VMEM_SHARED((acc_rows, D), jnp.float32),
        pltpu.VMEM((nch_w, K), jnp.int32),
        pltpu.VMEM((K, D), jnp.float32),
    ]
    if gather:
        scratch += [pltpu.VMEM((nch_w, K), jnp.int32), pltpu.SemaphoreType.DMA]

    @functools.partial(
        pl.kernel,
        out_type=jax.ShapeDtypeStruct((NC, acc_rows, D), jnp.float32),
        mesh=_mesh(),
        scratch_types=scratch,
    )
    def k(*refs):
        if gather:
            (y_hbm, srcc_hbm, dstc_hbm, zeros_hbm, out_hbm,
             acc, dst_v, rows_v, src_v, gsem) = refs
        else:
            (ones_hbm, dstc_hbm, zeros_hbm, out_hbm, acc, dst_v, rows_v) = refs
        cid = lax.axis_index("c")
        sid = lax.axis_index("s")
        w = cid * NS + sid
        pltpu.sync_copy(zeros_hbm, acc.at[pl.ds(sid * stripe, stripe)])
        pltpu.sync_copy(dstc_hbm.at[pl.ds(w * nch_w, nch_w)], dst_v)
        if gather:
            pltpu.sync_copy(srcc_hbm.at[pl.ds(w * nch_w, nch_w)], src_v)
        else:
            pltpu.sync_copy(ones_hbm, rows_v)
        plsc.subcore_barrier()

        if gather:
            def body(j, carry):
                pltpu.async_copy(y_hbm.at[src_v.at[j]], rows_v, gsem).wait()
                pltpu.sync_copy(rows_v, acc.at[dst_v.at[j]], add=True)
                return carry
        else:
            def body(j, carry):
                pltpu.sync_copy(rows_v, acc.at[dst_v.at[j]], add=True)
                return carry

        lax.fori_loop(0, nch_w, body, 0)
        plsc.subcore_barrier()
        pltpu.sync_copy(
            acc.at[pl.ds(sid * stripe, stripe)],
            out_hbm.at[cid, pl.ds(sid * stripe, stripe)],
        )

    return k


def _tc_scale_x(x, dega, degb, br):
    n, din = x.shape

    def body(x_ref, da_ref, db_ref, y_ref):
        deg = da_ref[...] + db_ref[...] + 1.0
        y_ref[...] = lax.rsqrt(deg) * x_ref[...]

    return pl.pallas_call(
        body,
        grid=(n // br,),
        in_specs=[
            pl.BlockSpec((br, din), lambda i: (i, 0)),
            pl.BlockSpec((br, 1), lambda i: (i, 0)),
            pl.BlockSpec((br, 1), lambda i: (i, 0)),
        ],
        out_specs=pl.BlockSpec((br, din), lambda i: (i, 0)),
        out_shape=jax.ShapeDtypeStruct((n, din), jnp.float32),
    )(x, dega, degb)


def _tc_mid(z1a, z1b, y1, dega, degb, w1, b1, w2p, br):
    n, din = y1.shape
    dh = w1.shape[1]
    d2 = w2p.shape[1]

    def body(za_ref, zb_ref, y1_ref, da_ref, db_ref, w1_ref, b1_ref, w2_ref, y2_ref):
        deg = da_ref[...] + db_ref[...] + 1.0
        dis = lax.rsqrt(deg)
        agg = dis * (za_ref[...] + zb_ref[...] + y1_ref[...])
        h = jnp.dot(agg, w1_ref[...], preferred_element_type=jnp.float32)
        h = jnp.maximum(h + b1_ref[...], 0.0)
        t = jnp.dot(h, w2_ref[...], preferred_element_type=jnp.float32)
        y2_ref[...] = dis * t

    return pl.pallas_call(
        body,
        grid=(n // br,),
        in_specs=[
            pl.BlockSpec((br, din), lambda i: (i, 0)),
            pl.BlockSpec((br, din), lambda i: (i, 0)),
            pl.BlockSpec((br, din), lambda i: (i, 0)),
            pl.BlockSpec((br, 1), lambda i: (i, 0)),
            pl.BlockSpec((br, 1), lambda i: (i, 0)),
            pl.BlockSpec((din, dh), lambda i: (0, 0)),
            pl.BlockSpec((1, dh), lambda i: (0, 0)),
            pl.BlockSpec((dh, d2), lambda i: (0, 0)),
        ],
        out_specs=pl.BlockSpec((br, d2), lambda i: (i, 0)),
        out_shape=jax.ShapeDtypeStruct((n, d2), jnp.float32),
    )(z1a, z1b, y1, dega, degb, w1, b1, w2p)


def _tc_final(z2a, z2b, y2, dega, degb, b2p, dc, br):
    n, d2 = y2.shape

    def body(za_ref, zb_ref, y2_ref, da_ref, db_ref, b2_ref, o_ref):
        deg = da_ref[...] + db_ref[...] + 1.0
        dis = lax.rsqrt(deg)
        v = dis * (za_ref[...] + zb_ref[...] + y2_ref[...]) + b2_ref[...]
        m = jnp.max(v, axis=1, keepdims=True)
        s = jnp.sum(jnp.exp(v - m), axis=1, keepdims=True)
        o_ref[...] = (v - m - jnp.log(s))[:, :dc]

    return pl.pallas_call(
        body,
        grid=(n // br,),
        in_specs=[
            pl.BlockSpec((br, d2), lambda i: (i, 0)),
            pl.BlockSpec((br, d2), lambda i: (i, 0)),
            pl.BlockSpec((br, d2), lambda i: (i, 0)),
            pl.BlockSpec((br, 1), lambda i: (i, 0)),
            pl.BlockSpec((br, 1), lambda i: (i, 0)),
            pl.BlockSpec((1, d2), lambda i: (0, 0)),
        ],
        out_specs=pl.BlockSpec((br, dc), lambda i: (i, 0)),
        out_shape=jax.ShapeDtypeStruct((n, dc), jnp.float32),
    )(z2a, z2b, y2, dega, degb, b2p)


def kernel(x, edge_index, W1, b1, W2, b2):
    n, din = x.shape
    e = edge_index.shape[1]
    dh = W1.shape[1]
    dc = W2.shape[1]
    br = 1000

    nw = NC * NS
    nch_w = -(-e // (K * nw * 8)) * 8
    cht = nch_w * nw
    epad = cht * K
    acc_rows = ((n + NS) + NS * 8 - 1) // (NS * 8) * (NS * 8)
    stripe = acc_rows // NS

    src = edge_index[0]
    dst = edge_index[1]
    pad = epad - e
    srcc = jnp.concatenate([src, jnp.zeros((pad,), jnp.int32)]).reshape(cht, K)
    dstc = jnp.concatenate([dst, jnp.full((pad,), n, jnp.int32)]).reshape(cht, K)

    onesKD = jnp.ones((K, D), jnp.float32)
    zerosD = jnp.zeros((stripe, D), jnp.float32)
    w2p = jnp.pad(W2, ((0, 0), (0, D - dc)))
    b2p = jnp.pad(b2, (0, D - dc), constant_values=-1e30).reshape(1, D)
    b1r = b1.reshape(1, dh)

    degp = _agg_kernel(acc_rows, nch_w, gather=False)(onesKD, dstc, zerosD)
    dega = degp[0, :n, 0:1]
    degb = degp[1, :n, 0:1]

    y1 = _tc_scale_x(x, dega, degb, br)

    z1p = _agg_kernel(acc_rows, nch_w, gather=True)(y1, srcc, dstc, zerosD)

    y2 = _tc_mid(z1p[0, :n, :], z1p[1, :n, :], y1, dega, degb, W1, b1r, w2p, br)

    z2p = _agg_kernel(acc_rows, nch_w, gather=True)(y2, srcc, dstc, zerosD)

    return _tc_final(z2p[0, :n, :], z2p[1, :n, :], y2, dega, degb, b2p, dc, br)
